# Initial kernel scaffold; baseline (speedup 1.0000x reference)
import jax
import jax.numpy as jnp
from jax.experimental import pallas as pl


def kernel(features_dis, edge_index_dis, edgenet_input_dis, features_hea, edge_index_hea, edgenet_input_hea, post_ind, nega_ind, pae_W1, pae_b1, pae_g, pae_beta, pae_W2, pae_b2, cheb0_W, cheb1_W, matrix_dis, enc_dis_W1, enc_dis_b1, enc_dis_g, enc_dis_beta, enc_dis_W2, enc_dis_b2, enc_hea_W1, enc_hea_b1, enc_hea_g, enc_hea_beta, enc_hea_W2, enc_hea_b2, cls_W1, cls_b1, cls_g, cls_beta, cls_W2, cls_b2):
    raise NotImplementedError("write your pallas kernel here")


# speedup vs baseline: 1.0043x; 1.0043x over previous
import functools

import jax
import jax.numpy as jnp
from jax.experimental import pallas as pl
from jax.experimental.pallas import tpu as pltpu

N = 10000
E = 320000
K = 6
GL = 0.25
TQ = 0.5
BN_EPS = 1e-5

_PAE_B = 8000


def _pae_body(e_ref, md_ref, W1_ref, b1_ref, g_ref, bt_ref, W2_ref, b2_ref,
              ew_ref, ss_ref):
    h = jnp.maximum(
        jnp.dot(e_ref[...], W1_ref[...], preferred_element_type=jnp.float32)
        + b1_ref[...], 0.0)
    h = h / jnp.sqrt(1.0 + BN_EPS) * g_ref[...] + bt_ref[...]
    w = jax.nn.sigmoid(
        jnp.dot(h, W2_ref[...], preferred_element_type=jnp.float32) + b2_ref[...])
    mdv = md_ref[...]
    ew0 = w - w * mdv

    @pl.when(pl.program_id(0) == 0)
    def _():
        ss_ref[...] = jnp.zeros_like(ss_ref)

    ss_ref[...] += jnp.stack(
        [jnp.sum(ew0 * ew0), jnp.sum(mdv * mdv)]).reshape(1, 2)
    ew_ref[...] = jnp.where(ew0 > GL, ew0, 0.0)


def _pae_edge_weights(eid_in, md, W1, b1, g, bt, W2, b2):
    grid = E // _PAE_B
    ew, ss = pl.pallas_call(
        _pae_body,
        grid=(grid,),
        in_specs=[
            pl.BlockSpec((_PAE_B, 4), lambda i: (i, 0)),
            pl.BlockSpec((_PAE_B, 1), lambda i: (i, 0)),
            pl.BlockSpec((4, 128), lambda i: (0, 0)),
            pl.BlockSpec((1, 128), lambda i: (0, 0)),
            pl.BlockSpec((1, 128), lambda i: (0, 0)),
            pl.BlockSpec((1, 128), lambda i: (0, 0)),
            pl.BlockSpec((128, 1), lambda i: (0, 0)),
            pl.BlockSpec((1, 1), lambda i: (0, 0)),
        ],
        out_specs=[
            pl.BlockSpec((_PAE_B, 1), lambda i: (i, 0)),
            pl.BlockSpec((1, 2), lambda i: (0, 0)),
        ],
        out_shape=[
            jax.ShapeDtypeStruct((E, 1), jnp.float32),
            jax.ShapeDtypeStruct((1, 2), jnp.float32),
        ],
    )(eid_in, md, W1, b1, g, bt, W2, b2)
    return ew, ss


def _bn_fold(g, beta, W2, b2):
    a = g / jnp.sqrt(1.0 + BN_EPS)
    return a[:, None] * W2, b2 + beta @ W2


def _cheb(x, src, dst, ew, W):
    deg = jax.ops.segment_sum(ew, src, num_segments=N)
    dinv = jnp.where(deg > 0.0, 1.0 / jnp.sqrt(jnp.where(deg > 0.0, deg, 1.0)), 0.0)
    lw = -(dinv[src] * ew * dinv[dst])

    def lmv(t):
        return jax.ops.segment_sum(lw[:, None] * t[src], dst, num_segments=N)

    tx0 = x
    out = tx0 @ W[0]
    tx1 = lmv(x)
    out = out + tx1 @ W[1]
    for k in range(2, K):
        tx2 = 2.0 * lmv(tx1) - tx0
        out = out + tx2 @ W[k]
        tx0, tx1 = tx1, tx2
    return out


def _mlp(x, W1, b1, W2f, b2f):
    h = jnp.maximum(x @ W1 + b1, 0.0)
    return h @ W2f + b2f


def _cos(a, b):
    na = jnp.maximum(jnp.sqrt(jnp.sum(a * a, axis=1)), 1e-8)
    nb = jnp.maximum(jnp.sqrt(jnp.sum(b * b, axis=1)), 1e-8)
    return jnp.sum(a * b, axis=1) / (na * nb)


def kernel(features_dis, edge_index_dis, edgenet_input_dis, features_hea, edge_index_hea, edgenet_input_hea, post_ind, nega_ind, pae_W1, pae_b1, pae_g, pae_beta, pae_W2, pae_b2, cheb0_W, cheb1_W, matrix_dis, enc_dis_W1, enc_dis_b1, enc_dis_g, enc_dis_beta, enc_dis_W2, enc_dis_b2, enc_hea_W1, enc_hea_b1, enc_hea_g, enc_hea_beta, enc_hea_W2, enc_hea_b2, cls_W1, cls_b1, cls_g, cls_beta, cls_W2, cls_b2):
    src = edge_index_dis[0]
    dst = edge_index_dis[1]

    ew2, ss = _pae_edge_weights(edgenet_input_dis, matrix_dis,
                                pae_W1, pae_b1[None, :], pae_g[None, :],
                                pae_beta[None, :], pae_W2, pae_b2[None, :])
    ew = jnp.squeeze(ew2, axis=1)
    norm = jnp.sqrt(ss[0, 0]) + jnp.sqrt(ss[0, 1])

    h = jnp.maximum(_cheb(features_dis, src, dst, ew, cheb0_W), 0.0)
    h0d = h
    h = jnp.maximum(_cheb(h, src, dst, ew, cheb1_W), 0.0)
    h0d = jnp.concatenate([h0d, h], axis=1)
    h = jnp.maximum(_cheb(features_hea, src, dst, ew, cheb0_W), 0.0)
    h0h = h
    h = jnp.maximum(_cheb(h, src, dst, ew, cheb1_W), 0.0)
    h0h = jnp.concatenate([h0h, h], axis=1)

    dW2f, db2f = _bn_fold(enc_dis_g, enc_dis_beta, enc_dis_W2, enc_dis_b2)
    hW2f, hb2f = _bn_fold(enc_hea_g, enc_hea_beta, enc_hea_W2, enc_hea_b2)
    kW2f, kb2f = _bn_fold(cls_g, cls_beta, cls_W2, cls_b2)

    d1 = _mlp(h0d, enc_dis_W1, enc_dis_b1, dW2f, db2f)
    d2 = _mlp(h0h, enc_hea_W1, enc_hea_b1, hW2f, hb2f)
    c = jnp.exp(_cos(d1, d2) / TQ)
    loss2 = -jnp.log(jnp.sum(c[post_ind]) / jnp.sum(c[nega_ind]))
    loss1 = jnp.sqrt(jnp.sum(enc_dis_W2 ** 2)) + jnp.sqrt(jnp.sum(enc_hea_W2 ** 2))
    go = jnp.concatenate([d1, d1 - d2], axis=1)
    logit = _mlp(go, cls_W1, cls_b1, kW2f, kb2f)
    return logit, norm, loss2, loss1


# speedup vs baseline: 2.0524x; 2.0436x over previous
import functools

import jax
import jax.numpy as jnp
from jax import lax
from jax.experimental import pallas as pl
from jax.experimental.pallas import tpu as pltpu
from jax.experimental.pallas import tpu_sc as plsc

N = 10000
E = 320000
K = 6
GL = 0.25
TQ = 0.5
BN_EPS = 1e-5

_PAE_B = 8000
_NTILES = 16
_CHUNK = 128
_CH = 160
_WJ = 16
_EP = _NTILES * _CH * _CHUNK
NP = 10240
_RT = NP // _NTILES
_RC = 128


def _pae_body(e_ref, md_ref, W1_ref, b1_ref, g_ref, bt_ref, W2_ref, b2_ref,
              ew_ref, ss_ref):
    h = jnp.maximum(
        jnp.dot(e_ref[...], W1_ref[...], preferred_element_type=jnp.float32)
        + b1_ref[...], 0.0)
    h = h / jnp.sqrt(1.0 + BN_EPS) * g_ref[...] + bt_ref[...]
    w = jax.nn.sigmoid(
        jnp.dot(h, W2_ref[...], preferred_element_type=jnp.float32) + b2_ref[...])
    mdv = md_ref[...]
    ew0 = w - w * mdv

    @pl.when(pl.program_id(0) == 0)
    def _():
        ss_ref[...] = jnp.zeros_like(ss_ref)

    ss_ref[...] += jnp.stack(
        [jnp.sum(ew0 * ew0), jnp.sum(mdv * mdv)]).reshape(1, 2)
    ew_ref[...] = jnp.where(ew0 > GL, ew0, 0.0)


def _pae_edge_weights(eid_in, md, W1, b1, g, bt, W2, b2):
    grid = E // _PAE_B
    ew, ss = pl.pallas_call(
        _pae_body,
        grid=(grid,),
        in_specs=[
            pl.BlockSpec((_PAE_B, 4), lambda i: (i, 0)),
            pl.BlockSpec((_PAE_B, 1), lambda i: (i, 0)),
            pl.BlockSpec((4, 128), lambda i: (0, 0)),
            pl.BlockSpec((1, 128), lambda i: (0, 0)),
            pl.BlockSpec((1, 128), lambda i: (0, 0)),
            pl.BlockSpec((1, 128), lambda i: (0, 0)),
            pl.BlockSpec((128, 1), lambda i: (0, 0)),
            pl.BlockSpec((1, 1), lambda i: (0, 0)),
        ],
        out_specs=[
            pl.BlockSpec((_PAE_B, 1), lambda i: (i, 0)),
            pl.BlockSpec((1, 2), lambda i: (0, 0)),
        ],
        out_shape=[
            jax.ShapeDtypeStruct((E, 1), jnp.float32),
            jax.ShapeDtypeStruct((1, 2), jnp.float32),
        ],
    )(eid_in, md, W1, b1, g, bt, W2, b2)
    return ew, ss


def _spmm_cheb(xs_all, srcs, dsts, lws, width):
    nv = width // 16
    mesh = plsc.VectorSubcoreMesh(core_axis_name="c", subcore_axis_name="s")
    out_types = [jax.ShapeDtypeStruct((2 * NP, width), jnp.float32)
                 for _ in range(5)]

    @functools.partial(
        pl.kernel,
        out_type=out_types,
        mesh=mesh,
        scratch_types=[
            pltpu.VMEM((_WJ, _CHUNK), jnp.int32),
            pltpu.VMEM((_WJ, _CHUNK), jnp.int32),
            pltpu.VMEM((_WJ, _CHUNK), jnp.float32),
            pltpu.VMEM((_CHUNK, width), jnp.float32),
            pltpu.VMEM((_RC, width), jnp.float32),
            pltpu.VMEM_SHARED((NP, width), jnp.float32),
            pltpu.SemaphoreType.DMA,
        ],
    )
    def k(xs_ref, srcs_ref, dsts_ref, lws_ref, t1, t2, t3, t4, t5,
          src_w, dst_w, lw_w, buf, a_buf, acc, sem):
        c = lax.axis_index("c")
        s = lax.axis_index("s")
        row0 = s * _RT
        gbase = c * NP + row0

        def _zero_abuf(r, _):
            for v in range(nv):
                a_buf[r, pl.ds(v * 16, 16)] = jnp.zeros((16,), jnp.float32)
            return 0

        lax.fori_loop(0, _RC, _zero_abuf, 0)
        for i in range(_RT // _RC):
            pltpu.sync_copy(a_buf, acc.at[pl.ds(row0 + i * _RC, _RC)])

        prevs = [xs_ref, t1, t2, t3, t4]
        prev2s = [None, xs_ref, t1, t2, t3]
        outs = [t1, t2, t3, t4, t5]

        for p in range(5):
            plsc.subcore_barrier()
            tprev = prevs[p]

            def _win(wi, _):
                pltpu.sync_copy(srcs_ref.at[c, s, pl.ds(wi * _WJ, _WJ)], src_w)
                pltpu.sync_copy(dsts_ref.at[s, pl.ds(wi * _WJ, _WJ)], dst_w)
                pltpu.sync_copy(lws_ref.at[s, pl.ds(wi * _WJ, _WJ)], lw_w)

                def _chunk(jj, _2):
                    pltpu.async_copy(tprev.at[src_w.at[jj]], buf, sem).wait()

                    def _group(g, _3):
                        lwv = lw_w[jj, pl.ds(g * 16, 16)]
                        for i in range(16):
                            e = g * 16 + i
                            sc = lwv[i]
                            for v in range(nv):
                                buf[e, pl.ds(v * 16, 16)] = (
                                    sc * buf[e, pl.ds(v * 16, 16)])
                        return 0

                    lax.fori_loop(0, _CHUNK // 16, _group, 0)
                    pltpu.sync_copy(buf, acc.at[dst_w.at[jj]], add=True)
                    return 0

                lax.fori_loop(0, _WJ, _chunk, 0)
                return 0

            lax.fori_loop(0, _CH // _WJ, _win, 0)
            plsc.subcore_barrier()

            tp = outs[p]
            for i in range(_RT // _RC):
                pltpu.sync_copy(acc.at[pl.ds(row0 + i * _RC, _RC)], a_buf)
                if p > 0:
                    pltpu.sync_copy(prev2s[p].at[pl.ds(gbase + i * _RC, _RC)],
                                    buf.at[pl.ds(0, _RC)])

                    def _rec(r, _):
                        for v in range(nv):
                            a_buf[r, pl.ds(v * 16, 16)] = (
                                2.0 * a_buf[r, pl.ds(v * 16, 16)]
                                - buf[r, pl.ds(v * 16, 16)])
                        return 0

                    lax.fori_loop(0, _RC, _rec, 0)
                pltpu.sync_copy(a_buf, tp.at[pl.ds(gbase + i * _RC, _RC)])
                if p < 4:
                    lax.fori_loop(0, _RC, _zero_abuf, 0)
                    pltpu.sync_copy(a_buf, acc.at[pl.ds(row0 + i * _RC, _RC)])

    return k(xs_all, srcs, dsts, lws)


def _pad_edges(v):
    return jnp.concatenate(
        [v, jnp.zeros((_EP - E,), v.dtype)]).reshape(_NTILES, _CH, _CHUNK)


def _bn_fold(g, beta, W2, b2):
    a = g / jnp.sqrt(1.0 + BN_EPS)
    return a[:, None] * W2, b2 + beta @ W2


def _mlp(x, W1, b1, W2f, b2f):
    h = jnp.maximum(x @ W1 + b1, 0.0)
    return h @ W2f + b2f


def _cos(a, b):
    na = jnp.maximum(jnp.sqrt(jnp.sum(a * a, axis=1)), 1e-8)
    nb = jnp.maximum(jnp.sqrt(jnp.sum(b * b, axis=1)), 1e-8)
    return jnp.sum(a * b, axis=1) / (na * nb)


def _cheb_layer(xs_all, srcs, dsts, lws, W):
    w = xs_all.shape[1]
    xsp = xs_all if w == 128 else jnp.concatenate(
        [xs_all, jnp.zeros((xs_all.shape[0], 128 - w), jnp.float32)], axis=1)
    ts = _spmm_cheb(xsp, srcs, dsts, lws, 128)
    out = xs_all @ W[0]
    for p in range(1, K):
        out = out + ts[p - 1][:, :w] @ W[p]
    return jnp.maximum(out, 0.0)


def kernel(features_dis, edge_index_dis, edgenet_input_dis, features_hea, edge_index_hea, edgenet_input_hea, post_ind, nega_ind, pae_W1, pae_b1, pae_g, pae_beta, pae_W2, pae_b2, cheb0_W, cheb1_W, matrix_dis, enc_dis_W1, enc_dis_b1, enc_dis_g, enc_dis_beta, enc_dis_W2, enc_dis_b2, enc_hea_W1, enc_hea_b1, enc_hea_g, enc_hea_beta, enc_hea_W2, enc_hea_b2, cls_W1, cls_b1, cls_g, cls_beta, cls_W2, cls_b2):
    src = edge_index_dis[0]
    dst = edge_index_dis[1]

    ew2, ss = _pae_edge_weights(edgenet_input_dis, matrix_dis,
                                pae_W1, pae_b1[None, :], pae_g[None, :],
                                pae_beta[None, :], pae_W2, pae_b2[None, :])
    ew = jnp.squeeze(ew2, axis=1)
    norm = jnp.sqrt(ss[0, 0]) + jnp.sqrt(ss[0, 1])

    deg = jax.ops.segment_sum(ew, src, num_segments=N)
    dinv = jnp.where(deg > 0.0, 1.0 / jnp.sqrt(jnp.where(deg > 0.0, deg, 1.0)), 0.0)
    lw = -(dinv[src] * ew * dinv[dst])

    srcp = _pad_edges(src.astype(jnp.int32))
    srcs = jnp.stack([srcp, srcp + NP])
    dsts = _pad_edges(dst.astype(jnp.int32))
    lws = _pad_edges(lw)

    zpad = jnp.zeros((NP - N, 128), jnp.float32)
    xs1 = jnp.concatenate([features_dis, zpad, features_hea, zpad], axis=0)
    h1 = _cheb_layer(xs1, srcs, dsts, lws, cheb0_W)
    h2 = _cheb_layer(h1, srcs, dsts, lws, cheb1_W)

    h0d = jnp.concatenate([h1[:N], h2[:N]], axis=1)
    h0h = jnp.concatenate([h1[NP:NP + N], h2[NP:NP + N]], axis=1)

    dW2f, db2f = _bn_fold(enc_dis_g, enc_dis_beta, enc_dis_W2, enc_dis_b2)
    hW2f, hb2f = _bn_fold(enc_hea_g, enc_hea_beta, enc_hea_W2, enc_hea_b2)
    kW2f, kb2f = _bn_fold(cls_g, cls_beta, cls_W2, cls_b2)

    d1 = _mlp(h0d, enc_dis_W1, enc_dis_b1, dW2f, db2f)
    d2 = _mlp(h0h, enc_hea_W1, enc_hea_b1, hW2f, hb2f)
    c = jnp.exp(_cos(d1, d2) / TQ)
    loss2 = -jnp.log(jnp.sum(c[post_ind]) / jnp.sum(c[nega_ind]))
    loss1 = jnp.sqrt(jnp.sum(enc_dis_W2 ** 2)) + jnp.sqrt(jnp.sum(enc_hea_W2 ** 2))
    go = jnp.concatenate([d1, d1 - d2], axis=1)
    logit = _mlp(go, cls_W1, cls_b1, kW2f, kb2f)
    return logit, norm, loss2, loss1


# speedup vs baseline: 2.2650x; 1.1036x over previous
import functools

import jax
import jax.numpy as jnp
from jax import lax
from jax.experimental import pallas as pl
from jax.experimental.pallas import tpu as pltpu
from jax.experimental.pallas import tpu_sc as plsc

N = 10000
E = 320000
K = 6
GL = 0.25
TQ = 0.5
BN_EPS = 1e-5

_PAE_B = 8000
_NTILES = 16
_CHUNK = 64
_CH = 320
_WCH = 32
_EP = _NTILES * _CH * _CHUNK
NP = 10240
_RT = NP // _NTILES
_RC = 64


def _pae_body(e_ref, md_ref, W1_ref, b1_ref, g_ref, bt_ref, W2_ref, b2_ref,
              ew_ref, ss_ref):
    h = jnp.maximum(
        jnp.dot(e_ref[...], W1_ref[...], preferred_element_type=jnp.float32)
        + b1_ref[...], 0.0)
    h = h / jnp.sqrt(1.0 + BN_EPS) * g_ref[...] + bt_ref[...]
    w = jax.nn.sigmoid(
        jnp.dot(h, W2_ref[...], preferred_element_type=jnp.float32) + b2_ref[...])
    mdv = md_ref[...]
    ew0 = w - w * mdv

    @pl.when(pl.program_id(0) == 0)
    def _():
        ss_ref[...] = jnp.zeros_like(ss_ref)

    ss_ref[...] += jnp.stack(
        [jnp.sum(ew0 * ew0), jnp.sum(mdv * mdv)]).reshape(1, 2)
    ew_ref[...] = jnp.where(ew0 > GL, ew0, 0.0)


def _pae_edge_weights(eid_in, md, W1, b1, g, bt, W2, b2):
    grid = E // _PAE_B
    ew, ss = pl.pallas_call(
        _pae_body,
        grid=(grid,),
        in_specs=[
            pl.BlockSpec((_PAE_B, 4), lambda i: (i, 0)),
            pl.BlockSpec((_PAE_B, 1), lambda i: (i, 0)),
            pl.BlockSpec((4, 128), lambda i: (0, 0)),
            pl.BlockSpec((1, 128), lambda i: (0, 0)),
            pl.BlockSpec((1, 128), lambda i: (0, 0)),
            pl.BlockSpec((1, 128), lambda i: (0, 0)),
            pl.BlockSpec((128, 1), lambda i: (0, 0)),
            pl.BlockSpec((1, 1), lambda i: (0, 0)),
        ],
        out_specs=[
            pl.BlockSpec((_PAE_B, 1), lambda i: (i, 0)),
            pl.BlockSpec((1, 2), lambda i: (0, 0)),
        ],
        out_shape=[
            jax.ShapeDtypeStruct((E, 1), jnp.float32),
            jax.ShapeDtypeStruct((1, 2), jnp.float32),
        ],
    )(eid_in, md, W1, b1, g, bt, W2, b2)
    return ew, ss


def _spmm_cheb(xs_all, srcs, dsts, lws, width):
    nv = width // 16
    mesh = plsc.VectorSubcoreMesh(core_axis_name="c", subcore_axis_name="s")
    out_types = [jax.ShapeDtypeStruct((2 * NP, width), jnp.float32)
                 for _ in range(5)]

    @functools.partial(
        pl.kernel,
        out_type=out_types,
        mesh=mesh,
        scratch_types=[
            pltpu.VMEM((_WCH, _CHUNK), jnp.int32),
            pltpu.VMEM((_WCH, _CHUNK), jnp.int32),
            pltpu.VMEM((_WCH, _CHUNK), jnp.float32),
            pltpu.VMEM((_CHUNK, width), jnp.float32),
            pltpu.VMEM((_CHUNK, width), jnp.float32),
            pltpu.VMEM((_RC, width), jnp.float32),
            pltpu.VMEM_SHARED((NP, width), jnp.float32),
            pltpu.SemaphoreType.DMA,
            pltpu.SemaphoreType.DMA,
            pltpu.SemaphoreType.DMA,
            pltpu.SemaphoreType.DMA,
        ],
    )
    def k(xs_ref, srcs_ref, dsts_ref, lws_ref, t1, t2, t3, t4, t5,
          src_w, dst_w, lw_w, buf_a, buf_b, a_buf, acc,
          gsem_a, gsem_b, ssem_a, ssem_b):
        c = lax.axis_index("c")
        s = lax.axis_index("s")
        row0 = s * _RT
        gbase = c * NP + row0

        def _zero_abuf(r, _):
            for v in range(nv):
                a_buf[r, pl.ds(v * 16, 16)] = jnp.zeros((16,), jnp.float32)
            return 0

        lax.fori_loop(0, _RC, _zero_abuf, 0)
        for i in range(_RT // _RC):
            pltpu.sync_copy(a_buf, acc.at[pl.ds(row0 + i * _RC, _RC)])

        prevs = [xs_ref, t1, t2, t3, t4]
        prev2s = [None, xs_ref, t1, t2, t3]
        outs = [t1, t2, t3, t4, t5]

        def _scale(buf, jj):
            def _group(g, _):
                lwv = lw_w[jj, pl.ds(g * 16, 16)]
                for i in range(16):
                    e = g * 16 + i
                    sc = lwv[i]
                    for v in range(nv):
                        buf[e, pl.ds(v * 16, 16)] = (
                            sc * buf[e, pl.ds(v * 16, 16)])
                return 0

            lax.fori_loop(0, _CHUNK // 16, _group, 0)

        for p in range(5):
            plsc.subcore_barrier()
            tprev = prevs[p]

            def _win(wi, _):
                pltpu.sync_copy(srcs_ref.at[c, s, pl.ds(wi * _WCH, _WCH)], src_w)
                pltpu.sync_copy(dsts_ref.at[s, pl.ds(wi * _WCH, _WCH)], dst_w)
                pltpu.sync_copy(lws_ref.at[s, pl.ds(wi * _WCH, _WCH)], lw_w)
                pltpu.async_copy(tprev.at[src_w.at[0]], buf_a, gsem_a)

                def _pair(st, _2):
                    ja = 2 * st
                    jb = ja + 1
                    pltpu.async_copy(tprev.at[src_w.at[jb]], buf_b, gsem_b)
                    pltpu.make_async_copy(
                        tprev.at[src_w.at[ja]], buf_a, gsem_a).wait()
                    _scale(buf_a, ja)
                    pltpu.async_copy(
                        buf_a, acc.at[dst_w.at[ja]], ssem_a, add=True)
                    pltpu.make_async_copy(
                        tprev.at[src_w.at[jb]], buf_b, gsem_b).wait()
                    _scale(buf_b, jb)
                    pltpu.make_async_copy(
                        buf_a, acc.at[dst_w.at[ja]], ssem_a).wait()
                    pltpu.async_copy(
                        buf_b, acc.at[dst_w.at[jb]], ssem_b, add=True)

                    @pl.when(st < _WCH // 2 - 1)
                    def _():
                        pltpu.async_copy(
                            tprev.at[src_w.at[ja + 2]], buf_a, gsem_a)

                    pltpu.make_async_copy(
                        buf_b, acc.at[dst_w.at[jb]], ssem_b).wait()
                    return 0

                lax.fori_loop(0, _WCH // 2, _pair, 0)
                return 0

            lax.fori_loop(0, _CH // _WCH, _win, 0)
            plsc.subcore_barrier()

            tp = outs[p]
            for i in range(_RT // _RC):
                pltpu.sync_copy(acc.at[pl.ds(row0 + i * _RC, _RC)], a_buf)
                if p > 0:
                    pltpu.sync_copy(prev2s[p].at[pl.ds(gbase + i * _RC, _RC)],
                                    buf_a.at[pl.ds(0, _RC)])

                    def _rec(r, _):
                        for v in range(nv):
                            a_buf[r, pl.ds(v * 16, 16)] = (
                                2.0 * a_buf[r, pl.ds(v * 16, 16)]
                                - buf_a[r, pl.ds(v * 16, 16)])
                        return 0

                    lax.fori_loop(0, _RC, _rec, 0)
                pltpu.sync_copy(a_buf, tp.at[pl.ds(gbase + i * _RC, _RC)])
                if p < 4:
                    lax.fori_loop(0, _RC, _zero_abuf, 0)
                    pltpu.sync_copy(a_buf, acc.at[pl.ds(row0 + i * _RC, _RC)])

    return k(xs_all, srcs, dsts, lws)


def _pad_edges(v):
    return jnp.concatenate(
        [v, jnp.zeros((_EP - E,), v.dtype)]).reshape(_NTILES, _CH, _CHUNK)


def _bn_fold(g, beta, W2, b2):
    a = g / jnp.sqrt(1.0 + BN_EPS)
    return a[:, None] * W2, b2 + beta @ W2


def _mlp(x, W1, b1, W2f, b2f):
    h = jnp.maximum(x @ W1 + b1, 0.0)
    return h @ W2f + b2f


def _cos(a, b):
    na = jnp.maximum(jnp.sqrt(jnp.sum(a * a, axis=1)), 1e-8)
    nb = jnp.maximum(jnp.sqrt(jnp.sum(b * b, axis=1)), 1e-8)
    return jnp.sum(a * b, axis=1) / (na * nb)


def _cheb_layer(xs_all, srcs, dsts, lws, W):
    w = xs_all.shape[1]
    xsp = xs_all if w == 128 else jnp.concatenate(
        [xs_all, jnp.zeros((xs_all.shape[0], 128 - w), jnp.float32)], axis=1)
    ts = _spmm_cheb(xsp, srcs, dsts, lws, 128)
    out = xs_all @ W[0]
    for p in range(1, K):
        out = out + ts[p - 1][:, :w] @ W[p]
    return jnp.maximum(out, 0.0)


def kernel(features_dis, edge_index_dis, edgenet_input_dis, features_hea, edge_index_hea, edgenet_input_hea, post_ind, nega_ind, pae_W1, pae_b1, pae_g, pae_beta, pae_W2, pae_b2, cheb0_W, cheb1_W, matrix_dis, enc_dis_W1, enc_dis_b1, enc_dis_g, enc_dis_beta, enc_dis_W2, enc_dis_b2, enc_hea_W1, enc_hea_b1, enc_hea_g, enc_hea_beta, enc_hea_W2, enc_hea_b2, cls_W1, cls_b1, cls_g, cls_beta, cls_W2, cls_b2):
    src = edge_index_dis[0]
    dst = edge_index_dis[1]

    ew2, ss = _pae_edge_weights(edgenet_input_dis, matrix_dis,
                                pae_W1, pae_b1[None, :], pae_g[None, :],
                                pae_beta[None, :], pae_W2, pae_b2[None, :])
    ew = jnp.squeeze(ew2, axis=1)
    norm = jnp.sqrt(ss[0, 0]) + jnp.sqrt(ss[0, 1])

    deg = jax.ops.segment_sum(ew, src, num_segments=N)
    dinv = jnp.where(deg > 0.0, 1.0 / jnp.sqrt(jnp.where(deg > 0.0, deg, 1.0)), 0.0)
    lw = -(dinv[src] * ew * dinv[dst])

    srcp = _pad_edges(src.astype(jnp.int32))
    srcs = jnp.stack([srcp, srcp + NP])
    dsts = _pad_edges(dst.astype(jnp.int32))
    lws = _pad_edges(lw)

    zpad = jnp.zeros((NP - N, 128), jnp.float32)
    xs1 = jnp.concatenate([features_dis, zpad, features_hea, zpad], axis=0)
    h1 = _cheb_layer(xs1, srcs, dsts, lws, cheb0_W)
    h2 = _cheb_layer(h1, srcs, dsts, lws, cheb1_W)

    h0d = jnp.concatenate([h1[:N], h2[:N]], axis=1)
    h0h = jnp.concatenate([h1[NP:NP + N], h2[NP:NP + N]], axis=1)

    dW2f, db2f = _bn_fold(enc_dis_g, enc_dis_beta, enc_dis_W2, enc_dis_b2)
    hW2f, hb2f = _bn_fold(enc_hea_g, enc_hea_beta, enc_hea_W2, enc_hea_b2)
    kW2f, kb2f = _bn_fold(cls_g, cls_beta, cls_W2, cls_b2)

    d1 = _mlp(h0d, enc_dis_W1, enc_dis_b1, dW2f, db2f)
    d2 = _mlp(h0h, enc_hea_W1, enc_hea_b1, hW2f, hb2f)
    c = jnp.exp(_cos(d1, d2) / TQ)
    loss2 = -jnp.log(jnp.sum(c[post_ind]) / jnp.sum(c[nega_ind]))
    loss1 = jnp.sqrt(jnp.sum(enc_dis_W2 ** 2)) + jnp.sqrt(jnp.sum(enc_hea_W2 ** 2))
    go = jnp.concatenate([d1, d1 - d2], axis=1)
    logit = _mlp(go, cls_W1, cls_b1, kW2f, kb2f)
    return logit, norm, loss2, loss1
